# Initial kernel scaffold; baseline (speedup 1.0000x reference)
import jax
import jax.numpy as jnp
from jax.experimental import pallas as pl


def kernel(x, edge_attr, t, p, params, edge_index, batch):
    raise NotImplementedError("write your pallas kernel here")


# speedup vs baseline: 5.0316x; 5.0316x over previous
import functools

import jax
import jax.numpy as jnp
from jax import lax
from jax.experimental import pallas as pl
from jax.experimental.pallas import tpu as pltpu
from jax.experimental.pallas import tpu_sc as plsc

N = 10000
E = 320000
IN = 128
H = 128
ED = 16
G = 512
NT = 2

NC = 2
NS = 16
L = 16
NW = NC * NS
EPW = E // NW
CH = 80
NCHUNK = EPW // CH
NPS = 624
NP = 80
ZB = 48

_mesh = plsc.VectorSubcoreMesh(
    core_axis_name="c", subcore_axis_name="s", num_cores=NC, num_subcores=NS)

EB = 2000
NEB = E // EB
NB = 1000
NNB = N // NB


def _lrelu(v):
    return jnp.where(v >= 0, v, 0.01 * v)


def _elu(v):
    return jnp.where(v > 0, v, jnp.exp(jnp.minimum(v, 0.0)) - 1.0)


def _gru(inp, hid, wit, bi, wht, bh):
    gi = jnp.dot(inp, wit, preferred_element_type=jnp.float32) + bi
    gh = jnp.dot(hid, wht, preferred_element_type=jnp.float32) + bh
    r = jax.nn.sigmoid(gi[:, :H] + gh[:, :H])
    z = jax.nn.sigmoid(gi[:, H:2 * H] + gh[:, H:2 * H])
    nn = jnp.tanh(gi[:, 2 * H:] + r * gh[:, 2 * H:])
    return (1.0 - z) * nn + z * hid


def _tca_body(x_ref, w1t, b1, gwat, gatr, x1_o, u_o, r_o):
    x1 = _lrelu(jnp.dot(x_ref[...], w1t[...],
                        preferred_element_type=jnp.float32) + b1[...])
    x1_o[...] = x1
    u_o[...] = jnp.dot(x1, gwat[...], preferred_element_type=jnp.float32)
    r_o[...] = jnp.sum(x1 * gatr[...], axis=1, keepdims=True)


def _tca(x, w1t, b1, gwat, gatr):
    full = lambda s: pl.BlockSpec(s, lambda i: (0, 0))
    return pl.pallas_call(
        _tca_body,
        grid=(NNB,),
        in_specs=[pl.BlockSpec((NB, IN), lambda i: (i, 0)),
                  full((IN, H)), full((1, H)), full((H, H)), full((1, H))],
        out_specs=[pl.BlockSpec((NB, H), lambda i: (i, 0)),
                   pl.BlockSpec((NB, H), lambda i: (i, 0)),
                   pl.BlockSpec((NB, 1), lambda i: (i, 0))],
        out_shape=[jax.ShapeDtypeStruct((N, H), jnp.float32),
                   jax.ShapeDtypeStruct((N, H), jnp.float32),
                   jax.ShapeDtypeStruct((N, 1), jnp.float32)],
    )(x, w1t, b1, gwat, gatr)


def _tcc_body(us_ref, ea_ref, rd_ref, gwbt, gl, m_o, lg_o, mx_o):
    i = pl.program_id(0)
    m = _lrelu(us_ref[...] + jnp.dot(ea_ref[...], gwbt[...],
                                     preferred_element_type=jnp.float32))
    m_o[...] = m
    tt = jnp.sum(m * gl[...], axis=1, keepdims=True)
    lg = _lrelu(tt + rd_ref[...])
    lg_o[...] = lg

    @pl.when(i == 0)
    def _():
        mx_o[...] = jnp.full((1, H), -1e30, jnp.float32)

    mx_o[...] = jnp.maximum(mx_o[...], jnp.max(lg))


def _tcc(usrc, ea, rdst, gwbt, gl):
    full = lambda s: pl.BlockSpec(s, lambda i: (0, 0))
    return pl.pallas_call(
        _tcc_body,
        grid=(NEB,),
        in_specs=[pl.BlockSpec((EB, H), lambda i: (i, 0)),
                  pl.BlockSpec((EB, ED), lambda i: (i, 0)),
                  pl.BlockSpec((EB, 1), lambda i: (i, 0)),
                  full((ED, H)), full((1, H))],
        out_specs=[pl.BlockSpec((EB, H), lambda i: (i, 0)),
                   pl.BlockSpec((EB, 1), lambda i: (i, 0)),
                   full((1, H))],
        out_shape=[jax.ShapeDtypeStruct((E, H), jnp.float32),
                   jax.ShapeDtypeStruct((E, 1), jnp.float32),
                   jax.ShapeDtypeStruct((1, H), jnp.float32)],
    )(usrc, ea, rdst, gwbt, gl)


def _tcmax_body(lg_ref, mx_o):
    i = pl.program_id(0)

    @pl.when(i == 0)
    def _():
        mx_o[...] = jnp.full((1, H), -1e30, jnp.float32)

    mx_o[...] = jnp.maximum(mx_o[...], jnp.max(lg_ref[...]))


def _tcmax(lg):
    return pl.pallas_call(
        _tcmax_body,
        grid=(NEB,),
        in_specs=[pl.BlockSpec((EB, 1), lambda i: (i, 0))],
        out_specs=pl.BlockSpec((1, H), lambda i: (0, 0)),
        out_shape=jax.ShapeDtypeStruct((1, H), jnp.float32),
    )(lg)


def _tcseg_body(lg_ref, dst_ref, mx_ref, p_o, sp_o):
    i = pl.program_id(0)
    pvals = jnp.exp(lg_ref[...] - mx_ref[0, 0])
    p_o[...] = pvals
    d = dst_ref[...]
    ddiv = jax.lax.shift_right_logical(d, 7)
    dmod = jax.lax.bitwise_and(d, 127)
    ohd = (ddiv == jax.lax.broadcasted_iota(jnp.int32, (1, NP), 1)
           ).astype(jnp.float32)
    ohm = (dmod == jax.lax.broadcasted_iota(jnp.int32, (1, H), 1)
           ).astype(jnp.float32)
    contrib = lax.dot_general(ohd, pvals * ohm, (((0,), (0,)), ((), ())),
                              preferred_element_type=jnp.float32)

    @pl.when(i == 0)
    def _():
        sp_o[...] = jnp.zeros((NP, H), jnp.float32)

    sp_o[...] += contrib


def _tcseg(lg, dst2d, mxvec):
    full = lambda s: pl.BlockSpec(s, lambda i: (0, 0))
    return pl.pallas_call(
        _tcseg_body,
        grid=(NEB,),
        in_specs=[pl.BlockSpec((EB, 1), lambda i: (i, 0)),
                  pl.BlockSpec((EB, 1), lambda i: (i, 0)),
                  full((1, H))],
        out_specs=[pl.BlockSpec((EB, 1), lambda i: (i, 0)),
                   full((NP, H))],
        out_shape=[jax.ShapeDtypeStruct((E, 1), jnp.float32),
                   jax.ShapeDtypeStruct((NP, H), jnp.float32)],
    )(lg, dst2d, mxvec)


def _tce_body(y_ref, s_ref, x1_ref, gw2t, gbias, wit, bi, wht, bh,
              awt, asrc, adst, x2_o, xl_o, as_o, ad_o):
    y = y_ref[0] + y_ref[1]
    h = _elu(jnp.dot(y / (s_ref[...] + 1e-16), gw2t[...],
                     preferred_element_type=jnp.float32) + gbias[...])
    x2 = jnp.maximum(_gru(h, x1_ref[...], wit[...], bi[...], wht[...],
                          bh[...]), 0.0)
    x2_o[...] = x2
    xl = jnp.dot(x2, awt[...], preferred_element_type=jnp.float32)
    xl_o[...] = xl
    as_o[...] = jnp.sum(xl * asrc[...], axis=1, keepdims=True)
    ad_o[...] = jnp.sum(xl * adst[...], axis=1, keepdims=True)


def _tce(y, s, x1, gw2t, gbias, wit, bi, wht, bh, awt, asrc, adst):
    full = lambda s_: pl.BlockSpec(s_, lambda i: tuple(0 for _ in s_))
    return pl.pallas_call(
        _tce_body,
        grid=(NNB,),
        in_specs=[pl.BlockSpec((NC, NB, H), lambda i: (0, i, 0)),
                  pl.BlockSpec((NB, 1), lambda i: (i, 0)),
                  pl.BlockSpec((NB, H), lambda i: (i, 0)),
                  full((H, H)), full((1, H)),
                  full((H, 3 * H)), full((1, 3 * H)),
                  full((H, 3 * H)), full((1, 3 * H)),
                  full((H, H)), full((1, H)), full((1, H))],
        out_specs=[pl.BlockSpec((NB, H), lambda i: (i, 0)),
                   pl.BlockSpec((NB, H), lambda i: (i, 0)),
                   pl.BlockSpec((NB, 1), lambda i: (i, 0)),
                   pl.BlockSpec((NB, 1), lambda i: (i, 0))],
        out_shape=[jax.ShapeDtypeStruct((N, H), jnp.float32),
                   jax.ShapeDtypeStruct((N, H), jnp.float32),
                   jax.ShapeDtypeStruct((N, 1), jnp.float32),
                   jax.ShapeDtypeStruct((N, 1), jnp.float32)],
    )(y, s, x1, gw2t, gbias, wit, bi, wht, bh, awt, asrc, adst)


def _tcj_body(y2_ref, s2_ref, x2_ref, b_ref,
              abias, awit, abi, awht, abh, mwst, mats,
              xs_o, as_o, pooled_o):
    i = pl.program_id(0)
    y = y2_ref[0] + y2_ref[1]
    h2 = _elu(y / (s2_ref[...] + 1e-16) + abias[...])
    x3 = jnp.maximum(_gru(h2, x2_ref[...], awit[...], abi[...],
                          awht[...], abh[...]), 0.0)
    xs = jnp.dot(x3, mwst[...], preferred_element_type=jnp.float32)
    xs_o[...] = xs
    as_o[...] = jnp.sum(xs * mats[...], axis=1, keepdims=True)
    oh = (b_ref[...] == jax.lax.broadcasted_iota(jnp.int32, (1, G), 1)
          ).astype(jnp.float32)

    @pl.when(i == 0)
    def _():
        pooled_o[...] = jnp.zeros((G, H), jnp.float32)

    pooled_o[...] += lax.dot_general(oh, x3, (((0,), (0,)), ((), ())),
                                     preferred_element_type=jnp.float32)


def _tcj(y2, s2, x2, batch2d, abias, awit, abi, awht, abh, mwst, mats):
    full = lambda s_: pl.BlockSpec(s_, lambda i: tuple(0 for _ in s_))
    return pl.pallas_call(
        _tcj_body,
        grid=(NNB,),
        in_specs=[pl.BlockSpec((NC, NB, H), lambda i: (0, i, 0)),
                  pl.BlockSpec((NB, 1), lambda i: (i, 0)),
                  pl.BlockSpec((NB, H), lambda i: (i, 0)),
                  pl.BlockSpec((NB, 1), lambda i: (i, 0)),
                  full((1, H)),
                  full((H, 3 * H)), full((1, 3 * H)),
                  full((H, 3 * H)), full((1, 3 * H)),
                  full((H, H)), full((1, H))],
        out_specs=[pl.BlockSpec((NB, H), lambda i: (i, 0)),
                   pl.BlockSpec((NB, 1), lambda i: (i, 0)),
                   full((G, H))],
        out_shape=[jax.ShapeDtypeStruct((N, H), jnp.float32),
                   jax.ShapeDtypeStruct((N, 1), jnp.float32),
                   jax.ShapeDtypeStruct((G, H), jnp.float32)],
    )(y2, s2, x2, batch2d, abias, awit, abi, awht, abh, mwst, mats)


def _tci_body(xs_ref, as_ref, b_ref, pooled_ref, t_ref, pf_ref,
              mwdt, matd, mbias, mwit, mbi, mwht, mbh,
              l2t, l2b, l3t, l3b, l4t, l4b,
              out_o, lg_s):
    iota_g = jax.lax.broadcasted_iota(jnp.int32, (1, G), 1)
    out = jnp.maximum(pooled_ref[...], 0.0)
    for _ in range(NT):
        od = jnp.dot(out, mwdt[...], preferred_element_type=jnp.float32)
        a_d = jnp.sum(od * matd[...], axis=1, keepdims=True)
        mseg = jnp.full((1, G), -1e30, jnp.float32)
        for b in range(NNB):
            sl = pl.ds(b * NB, NB)
            oh = (b_ref[sl, :] == iota_g).astype(jnp.float32)
            adn = jnp.dot(oh, a_d, preferred_element_type=jnp.float32,
                          precision=lax.Precision.HIGHEST)
            lgb = _lrelu(as_ref[sl, :] + adn)
            lg_s[sl, :] = lgb
            masked = jnp.where(oh > 0, lgb, -1e30)
            mseg = jnp.maximum(mseg, jnp.max(masked, axis=0, keepdims=True))
        mseg = jnp.where(mseg > -1e29, mseg, 0.0)
        sg = jnp.zeros((G, 1), jnp.float32)
        wsum = jnp.zeros((G, H), jnp.float32)
        for b in range(NNB):
            sl = pl.ds(b * NB, NB)
            oh = (b_ref[sl, :] == iota_g).astype(jnp.float32)
            mnode = jnp.dot(oh, mseg.T, preferred_element_type=jnp.float32,
                            precision=lax.Precision.HIGHEST)
            pb = jnp.exp(lg_s[sl, :] - mnode)
            sg = sg + lax.dot_general(oh, pb, (((0,), (0,)), ((), ())),
                                     preferred_element_type=jnp.float32,
                                     precision=lax.Precision.HIGHEST)
            wsum = wsum + lax.dot_general(
                oh, xs_ref[sl, :] * pb, (((0,), (0,)), ((), ())),
                preferred_element_type=jnp.float32)
        h = _elu(wsum / (sg + 1e-16) + mbias[...])
        out = jnp.maximum(_gru(h, out, mwit[...], mbi[...], mwht[...],
                               mbh[...]), 0.0)
    z = jnp.concatenate([out, t_ref[...], pf_ref[...]], axis=1)
    z = jnp.maximum(jnp.dot(z, l2t[...],
                            preferred_element_type=jnp.float32) + l2b[...],
                    0.0)
    z = jnp.maximum(jnp.dot(z, l3t[...],
                            preferred_element_type=jnp.float32) + l3b[...],
                    0.0)
    out_o[...] = jnp.dot(z, l4t[...],
                         preferred_element_type=jnp.float32) + l4b[...]


def _tci(xs, as_, batch2d, pooled, t, pf, ws):
    return pl.pallas_call(
        _tci_body,
        out_shape=jax.ShapeDtypeStruct((G, 1), jnp.float32),
        scratch_shapes=[pltpu.VMEM((N, 1), jnp.float32)],
        compiler_params=pltpu.CompilerParams(
            vmem_limit_bytes=60 * 1024 * 1024),
    )(xs, as_, batch2d, pooled, t, pf, *ws)


def _stage_table(hbm, sh, sid):
    pltpu.sync_copy(hbm.at[pl.ds(sid * NPS, NPS)],
                    sh.at[pl.ds(sid * NPS, NPS)])

    @pl.when(sid == NS - 1)
    def _():
        pltpu.sync_copy(hbm.at[pl.ds(NS * NPS, N - NS * NPS)],
                        sh.at[pl.ds(NS * NPS, N - NS * NPS)])


def _stage_pack(hbm, sh, sid):
    @pl.when(sid < NP // 8)
    def _():
        pltpu.sync_copy(hbm.at[pl.ds(sid * 8, 8)], sh.at[pl.ds(sid * 8, 8)])


def _dump_table(acc_sh, y_hbm, cid, sid):
    pltpu.sync_copy(acc_sh.at[pl.ds(sid * NPS, NPS)],
                    y_hbm.at[cid, pl.ds(sid * NPS, NPS)])

    @pl.when(sid == NS - 1)
    def _():
        pltpu.sync_copy(acc_sh.at[pl.ds(NS * NPS, N - NS * NPS)],
                        y_hbm.at[cid, pl.ds(NS * NPS, N - NS * NPS)])

def _scalar_extract(rows_v, dm, g, acc0):
    def body(j, acc):
        lane_oh = jax.lax.broadcasted_iota(jnp.int32, (L,), 0) == j
        dmj = jnp.sum(jnp.where(lane_oh, dm.astype(jnp.float32), 0.0)
                      ).astype(jnp.int32)
        hi = pl.multiple_of(jax.lax.bitwise_and(dmj, 112), L)
        lo = jax.lax.bitwise_and(dmj, 15)
        vv = rows_v[g * L + j, pl.ds(hi, L)]
        sp = vv[jnp.full((L,), lo, jnp.int32)]
        return acc + jnp.where(lane_oh, sp, 0.0)
    return lax.fori_loop(0, L, body, acc0)


def _scb_body(u_hbm, rp_hbm, src_hbm, dst_hbm, us_hbm, rd_hbm,
              utab_sh, rtab_sh, sidx_v, didx_v, ddiv_v, urows_v, rrows_v,
              rout_v, sem):
    cid = lax.axis_index("c")
    sid = lax.axis_index("s")
    wid = sid * NC + cid
    _stage_table(u_hbm, utab_sh, sid)
    _stage_pack(rp_hbm, rtab_sh, sid)
    plsc.subcore_barrier()
    base = wid * EPW

    def chunk(c, _):
        cb = base + c * CH
        pltpu.sync_copy(src_hbm.at[pl.ds(cb, CH)], sidx_v)
        pltpu.sync_copy(dst_hbm.at[pl.ds(cb, CH)], didx_v)
        pltpu.async_copy(utab_sh.at[sidx_v], urows_v, sem).wait()
        pltpu.sync_copy(urows_v, us_hbm.at[pl.ds(cb, CH)])

        def div_grp(g, _):
            dv = didx_v[pl.ds(g * L, L)]
            ddiv_v[pl.ds(g * L, L)] = jax.lax.shift_right_logical(dv, 7)
            return 0
        lax.fori_loop(0, CH // L, div_grp, 0)
        pltpu.async_copy(rtab_sh.at[ddiv_v], rrows_v, sem).wait()

        def ext_grp(g, _):
            dm = jax.lax.bitwise_and(didx_v[pl.ds(g * L, L)], 127)
            rout_v[pl.ds(g * L, L)] = _scalar_extract(
                rrows_v, dm, g, jnp.zeros((L,), jnp.float32))
            return 0
        lax.fori_loop(0, CH // L, ext_grp, 0)
        pltpu.sync_copy(rout_v, rd_hbm.at[pl.ds(cb, CH)])
        return 0
    lax.fori_loop(0, NCHUNK, chunk, 0)


def _scb(u, rpack, src, dst):
    return pl.kernel(
        _scb_body,
        out_type=(jax.ShapeDtypeStruct((E, H), jnp.float32),
                  jax.ShapeDtypeStruct((E,), jnp.float32)),
        mesh=_mesh,
        compiler_params=pltpu.CompilerParams(needs_layout_passes=False),
        scratch_types=[
            pltpu.VMEM_SHARED((N, H), jnp.float32),
            pltpu.VMEM_SHARED((NP, H), jnp.float32),
            pltpu.VMEM((CH,), jnp.int32),
            pltpu.VMEM((CH,), jnp.int32),
            pltpu.VMEM((CH,), jnp.int32),
            pltpu.VMEM((CH, H), jnp.float32),
            pltpu.VMEM((CH, H), jnp.float32),
            pltpu.VMEM((CH,), jnp.float32),
            pltpu.SemaphoreType.DMA,
        ],
    )(u, rpack, src, dst)


def _zero_spmem_slice(acc_sh, zbuf_v, sid):
    def zrow(i, _):
        def zcol(k, _):
            zbuf_v[i, pl.ds(k * L, L)] = jnp.zeros((L,), jnp.float32)
            return 0
        lax.fori_loop(0, H // L, zcol, 0)
        return 0
    lax.fori_loop(0, ZB, zrow, 0)

    def zcopy(k, _):
        pltpu.sync_copy(zbuf_v, acc_sh.at[pl.ds(sid * NPS + k * ZB, ZB)])
        return 0
    lax.fori_loop(0, NPS // ZB, zcopy, 0)

    @pl.when(sid == NS - 1)
    def _():
        pltpu.sync_copy(zbuf_v.at[pl.ds(0, N - NS * NPS)],
                        acc_sh.at[pl.ds(NS * NPS, N - NS * NPS)])


def _scale_rows(rows_v, p_v, ybuf_v):
    def grp(g, _):
        pvec = p_v[pl.ds(g * L, L)]

        def edge(j, _):
            sp = pvec[jnp.full((L,), j, jnp.int32)]
            e = g * L + j

            def col(k, _):
                ybuf_v[e, pl.ds(k * L, L)] = rows_v[e, pl.ds(k * L, L)] * sp
                return 0
            lax.fori_loop(0, H // L, col, 0)
            return 0
        lax.fori_loop(0, L, edge, 0)
        return 0
    lax.fori_loop(0, CH // L, grp, 0)


def _scd_body(m_hbm, p_hbm, dst_hbm, y_hbm,
              acc_sh, didx_v, p_v, mrows_v, ybuf_v, zbuf_v, sem):
    cid = lax.axis_index("c")
    sid = lax.axis_index("s")
    wid = sid * NC + cid
    _zero_spmem_slice(acc_sh, zbuf_v, sid)
    plsc.subcore_barrier()
    base = wid * EPW

    def chunk(c, _):
        cb = base + c * CH
        pltpu.sync_copy(dst_hbm.at[pl.ds(cb, CH)], didx_v)
        pltpu.sync_copy(p_hbm.at[pl.ds(cb, CH)], p_v)
        pltpu.sync_copy(m_hbm.at[pl.ds(cb, CH)], mrows_v)
        _scale_rows(mrows_v, p_v, ybuf_v)
        pltpu.sync_copy(ybuf_v, acc_sh.at[didx_v], add=True)
        return 0
    lax.fori_loop(0, NCHUNK, chunk, 0)
    plsc.subcore_barrier()
    _dump_table(acc_sh, y_hbm, cid, sid)


def _scd(m, pvals, dst):
    return pl.kernel(
        _scd_body,
        out_type=jax.ShapeDtypeStruct((NC, N, H), jnp.float32),
        mesh=_mesh,
        compiler_params=pltpu.CompilerParams(needs_layout_passes=False),
        scratch_types=[
            pltpu.VMEM_SHARED((N, H), jnp.float32),
            pltpu.VMEM((CH,), jnp.int32),
            pltpu.VMEM((CH,), jnp.float32),
            pltpu.VMEM((CH, H), jnp.float32),
            pltpu.VMEM((CH, H), jnp.float32),
            pltpu.VMEM((ZB, H), jnp.float32),
            pltpu.SemaphoreType.DMA,
        ],
    )(m, pvals, dst)


def _scf_body(ap_hbm, dp_hbm, src_hbm, dst_hbm, lg_hbm,
              atab_sh, dtab_sh, sidx_v, didx_v, sdiv_v, ddiv_v,
              arows_v, drows_v, lout_v, sem):
    cid = lax.axis_index("c")
    sid = lax.axis_index("s")
    wid = sid * NC + cid
    _stage_pack(ap_hbm, atab_sh, sid)
    _stage_pack(dp_hbm, dtab_sh, sid)
    plsc.subcore_barrier()
    base = wid * EPW

    def chunk(c, _):
        cb = base + c * CH
        pltpu.sync_copy(src_hbm.at[pl.ds(cb, CH)], sidx_v)
        pltpu.sync_copy(dst_hbm.at[pl.ds(cb, CH)], didx_v)

        def div_grp(g, _):
            sidx = sidx_v[pl.ds(g * L, L)]
            didx = didx_v[pl.ds(g * L, L)]
            sdiv_v[pl.ds(g * L, L)] = jax.lax.shift_right_logical(sidx, 7)
            ddiv_v[pl.ds(g * L, L)] = jax.lax.shift_right_logical(didx, 7)
            return 0
        lax.fori_loop(0, CH // L, div_grp, 0)
        pltpu.async_copy(atab_sh.at[sdiv_v], arows_v, sem).wait()
        pltpu.async_copy(dtab_sh.at[ddiv_v], drows_v, sem).wait()

        def ext_grp(g, _):
            sm = jax.lax.bitwise_and(sidx_v[pl.ds(g * L, L)], 127)
            dm = jax.lax.bitwise_and(didx_v[pl.ds(g * L, L)], 127)
            va = _scalar_extract(arows_v, sm, g, jnp.zeros((L,), jnp.float32))
            vd = _scalar_extract(drows_v, dm, g, jnp.zeros((L,), jnp.float32))
            lout_v[pl.ds(g * L, L)] = _lrelu(va + vd)
            return 0
        lax.fori_loop(0, CH // L, ext_grp, 0)
        pltpu.sync_copy(lout_v, lg_hbm.at[pl.ds(cb, CH)])
        return 0
    lax.fori_loop(0, NCHUNK, chunk, 0)


def _scf(aspack, adpack, src, dst):
    return pl.kernel(
        _scf_body,
        out_type=jax.ShapeDtypeStruct((E,), jnp.float32),
        mesh=_mesh,
        compiler_params=pltpu.CompilerParams(needs_layout_passes=False),
        scratch_types=[
            pltpu.VMEM_SHARED((NP, H), jnp.float32),
            pltpu.VMEM_SHARED((NP, H), jnp.float32),
            pltpu.VMEM((CH,), jnp.int32),
            pltpu.VMEM((CH,), jnp.int32),
            pltpu.VMEM((CH,), jnp.int32),
            pltpu.VMEM((CH,), jnp.int32),
            pltpu.VMEM((CH, H), jnp.float32),
            pltpu.VMEM((CH, H), jnp.float32),
            pltpu.VMEM((CH,), jnp.float32),
            pltpu.SemaphoreType.DMA,
        ],
    )(aspack, adpack, src, dst)


def _sch_body(xl_hbm, p_hbm, src_hbm, dst_hbm, y_hbm,
              acc_sh, sidx_v, didx_v, p_v, xrows_v, ybuf_v, zbuf_v, sem):
    cid = lax.axis_index("c")
    sid = lax.axis_index("s")
    wid = sid * NC + cid
    _zero_spmem_slice(acc_sh, zbuf_v, sid)
    plsc.subcore_barrier()
    base = wid * EPW

    def chunk(c, _):
        cb = base + c * CH
        pltpu.sync_copy(src_hbm.at[pl.ds(cb, CH)], sidx_v)
        pltpu.sync_copy(dst_hbm.at[pl.ds(cb, CH)], didx_v)
        pltpu.sync_copy(p_hbm.at[pl.ds(cb, CH)], p_v)
        pltpu.async_copy(xl_hbm.at[sidx_v], xrows_v, sem).wait()
        _scale_rows(xrows_v, p_v, ybuf_v)
        pltpu.sync_copy(ybuf_v, acc_sh.at[didx_v], add=True)
        return 0
    lax.fori_loop(0, NCHUNK, chunk, 0)
    plsc.subcore_barrier()
    _dump_table(acc_sh, y_hbm, cid, sid)


def _sch(xl, pvals, src, dst):
    return pl.kernel(
        _sch_body,
        out_type=jax.ShapeDtypeStruct((NC, N, H), jnp.float32),
        mesh=_mesh,
        compiler_params=pltpu.CompilerParams(needs_layout_passes=False),
        scratch_types=[
            pltpu.VMEM_SHARED((N, H), jnp.float32),
            pltpu.VMEM((CH,), jnp.int32),
            pltpu.VMEM((CH,), jnp.int32),
            pltpu.VMEM((CH,), jnp.float32),
            pltpu.VMEM((CH, H), jnp.float32),
            pltpu.VMEM((CH, H), jnp.float32),
            pltpu.VMEM((ZB, H), jnp.float32),
            pltpu.SemaphoreType.DMA,
        ],
    )(xl, pvals, src, dst)


def _pack_scalar(v):
    return jnp.pad(v.reshape(-1), (0, NP * H - N)).reshape(NP, H)


def kernel(x, edge_attr, t, p, params, edge_index, batch):
    pr = params
    src = edge_index[0]
    dst = edge_index[1]

    x1, u, r = _tca(x, pr['lin1_w'].T, pr['lin1_b'][None],
                    pr['g_lin1_w'][:, :IN].T, pr['g_att_r'][None])
    usrc, rdst = _scb(u, _pack_scalar(r), src, dst)
    m, logit, mxvec = _tcc(usrc, edge_attr, rdst[:, None],
                           pr['g_lin1_w'][:, IN:].T, pr['g_att_l'][None])
    pvals, spack = _tcseg(logit, dst[:, None], mxvec)
    y = _scd(m, pvals.reshape(-1), dst)
    s = spack.reshape(-1)[:N][:, None]
    x2, xl, as_, ad_ = _tce(
        y, s, x1, pr['g_lin2_w'].T, pr['g_bias'][None],
        pr['gru1_wi'].T, pr['gru1_bi'][None], pr['gru1_wh'].T,
        pr['gru1_bh'][None], pr['a_w'].T, pr['a_att_src'][None],
        pr['a_att_dst'][None])
    lg2 = _scf(_pack_scalar(as_), _pack_scalar(ad_), src, dst)
    mx2vec = _tcmax(lg2[:, None])
    p2, s2pack = _tcseg(lg2[:, None], dst[:, None], mx2vec)
    y2 = _sch(xl, p2.reshape(-1), src, dst)
    s2 = s2pack.reshape(-1)[:N][:, None]
    xs, a_s, pooled = _tcj(
        y2, s2, x2, batch[:, None], pr['a_bias'][None],
        pr['agru_wi'].T, pr['agru_bi'][None], pr['agru_wh'].T,
        pr['agru_bh'][None], pr['m_w_src'].T, pr['m_att_src'][None])
    ws = [pr['m_w_dst'].T, pr['m_att_dst'][None], pr['m_bias'][None],
          pr['mgru_wi'].T, pr['mgru_bi'][None], pr['mgru_wh'].T,
          pr['mgru_bh'][None],
          pr['lin2_w'].T, pr['lin2_b'][None],
          pr['lin3_w'].T, pr['lin3_b'][None],
          pr['lin4_w'].T, pr['lin4_b'][None]]
    out = _tci(xs, a_s, batch[:, None], pooled, t, p, ws)
    return out.reshape(-1)


# speedup vs baseline: 6.5770x; 1.3071x over previous
import functools

import jax
import jax.numpy as jnp
from jax import lax
from jax.experimental import pallas as pl
from jax.experimental.pallas import tpu as pltpu
from jax.experimental.pallas import tpu_sc as plsc

N = 10000
E = 320000
IN = 128
H = 128
ED = 16
G = 512
NT = 2

NC = 2
NS = 16
L = 16
NW = NC * NS
EPW = E // NW
CH = 80
NCHUNK = EPW // CH
NPS = 624
NP = 80
ZB = 48

_mesh = plsc.VectorSubcoreMesh(
    core_axis_name="c", subcore_axis_name="s", num_cores=NC, num_subcores=NS)

EB = 2000
NEB = E // EB
NB = 1000
NNB = N // NB


def _lrelu(v):
    return jnp.where(v >= 0, v, 0.01 * v)


def _elu(v):
    return jnp.where(v > 0, v, jnp.exp(jnp.minimum(v, 0.0)) - 1.0)


def _gru(inp, hid, wit, bi, wht, bh):
    gi = jnp.dot(inp, wit, preferred_element_type=jnp.float32) + bi
    gh = jnp.dot(hid, wht, preferred_element_type=jnp.float32) + bh
    r = jax.nn.sigmoid(gi[:, :H] + gh[:, :H])
    z = jax.nn.sigmoid(gi[:, H:2 * H] + gh[:, H:2 * H])
    nn = jnp.tanh(gi[:, 2 * H:] + r * gh[:, 2 * H:])
    return (1.0 - z) * nn + z * hid


def _tca_body(x_ref, w1t, b1, gatr, x1_o, r_o):
    x1 = _lrelu(jnp.dot(x_ref[...], w1t[...],
                        preferred_element_type=jnp.float32) + b1[...])
    x1_o[...] = x1
    r_o[...] = jnp.sum(x1 * gatr[...], axis=1, keepdims=True)


def _tca(x, w1t, b1, gatr):
    full = lambda s: pl.BlockSpec(s, lambda i: (0, 0))
    return pl.pallas_call(
        _tca_body,
        grid=(NNB,),
        in_specs=[pl.BlockSpec((NB, IN), lambda i: (i, 0)),
                  full((IN, H)), full((1, H)), full((1, H))],
        out_specs=[pl.BlockSpec((NB, H), lambda i: (i, 0)),
                   pl.BlockSpec((NB, 1), lambda i: (i, 0))],
        out_shape=[jax.ShapeDtypeStruct((N, H), jnp.float32),
                   jax.ShapeDtypeStruct((N, 1), jnp.float32)],
    )(x, w1t, b1, gatr)


def _tcc_body(us_ref, ea_ref, rd_ref, gw1t, gw2t, gl, m_o, lg_o, mx_o):
    i = pl.program_id(0)
    cat = jnp.concatenate([us_ref[...], ea_ref[...]], axis=1)
    m = _lrelu(jnp.dot(cat, gw1t[...], preferred_element_type=jnp.float32))
    m_o[...] = jnp.dot(m, gw2t[...], preferred_element_type=jnp.float32)
    tt = jnp.sum(m * gl[...], axis=1, keepdims=True)
    lg = _lrelu(tt + rd_ref[...])
    lg_o[...] = lg

    @pl.when(i == 0)
    def _():
        mx_o[...] = jnp.full((1, H), -1e30, jnp.float32)

    mx_o[...] = jnp.maximum(mx_o[...], jnp.max(lg))


def _tcc(usrc, ea, rdst, gw1t, gw2t, gl):
    full = lambda s: pl.BlockSpec(s, lambda i: (0, 0))
    return pl.pallas_call(
        _tcc_body,
        grid=(NEB,),
        in_specs=[pl.BlockSpec((EB, H), lambda i: (i, 0)),
                  pl.BlockSpec((EB, ED), lambda i: (i, 0)),
                  pl.BlockSpec((EB, 1), lambda i: (i, 0)),
                  full((H + ED, H)), full((H, H)), full((1, H))],
        out_specs=[pl.BlockSpec((EB, H), lambda i: (i, 0)),
                   pl.BlockSpec((EB, 1), lambda i: (i, 0)),
                   full((1, H))],
        out_shape=[jax.ShapeDtypeStruct((E, H), jnp.float32),
                   jax.ShapeDtypeStruct((E, 1), jnp.float32),
                   jax.ShapeDtypeStruct((1, H), jnp.float32)],
    )(usrc, ea, rdst, gw1t, gw2t, gl)


def _tcmax_body(lg_ref, mx_o):
    i = pl.program_id(0)

    @pl.when(i == 0)
    def _():
        mx_o[...] = jnp.full((1, H), -1e30, jnp.float32)

    mx_o[...] = jnp.maximum(mx_o[...], jnp.max(lg_ref[...]))


def _tcmax(lg):
    return pl.pallas_call(
        _tcmax_body,
        grid=(NEB,),
        in_specs=[pl.BlockSpec((EB, 1), lambda i: (i, 0))],
        out_specs=pl.BlockSpec((1, H), lambda i: (0, 0)),
        out_shape=jax.ShapeDtypeStruct((1, H), jnp.float32),
    )(lg)


def _tcseg_body(lg_ref, dst_ref, mx_ref, p_o, sp_o):
    i = pl.program_id(0)
    pvals = jnp.exp(lg_ref[...] - mx_ref[0, 0])
    p_o[...] = pvals
    d = dst_ref[...]
    ddiv = jax.lax.shift_right_logical(d, 7)
    dmod = jax.lax.bitwise_and(d, 127)
    ohd = (ddiv == jax.lax.broadcasted_iota(jnp.int32, (1, NP), 1)
           ).astype(jnp.float32)
    ohm = (dmod == jax.lax.broadcasted_iota(jnp.int32, (1, H), 1)
           ).astype(jnp.float32)
    contrib = lax.dot_general(ohd, pvals * ohm, (((0,), (0,)), ((), ())),
                              preferred_element_type=jnp.float32,
                              precision=lax.Precision.HIGHEST)

    @pl.when(i == 0)
    def _():
        sp_o[...] = jnp.zeros((NP, H), jnp.float32)

    sp_o[...] += contrib


def _tcseg(lg, dst2d, mxvec):
    full = lambda s: pl.BlockSpec(s, lambda i: (0, 0))
    return pl.pallas_call(
        _tcseg_body,
        grid=(NEB,),
        in_specs=[pl.BlockSpec((EB, 1), lambda i: (i, 0)),
                  pl.BlockSpec((EB, 1), lambda i: (i, 0)),
                  full((1, H))],
        out_specs=[pl.BlockSpec((EB, 1), lambda i: (i, 0)),
                   full((NP, H))],
        out_shape=[jax.ShapeDtypeStruct((E, 1), jnp.float32),
                   jax.ShapeDtypeStruct((NP, H), jnp.float32)],
    )(lg, dst2d, mxvec)


def _tce_body(y_ref, s_ref, x1_ref, gbias, wit, bi, wht, bh,
              awt, asrc, adst, x2_o, xl_o, as_o, ad_o):
    y = y_ref[0] + y_ref[1]
    h = _elu(y / (s_ref[...] + 1e-16) + gbias[...])
    x2 = jnp.maximum(_gru(h, x1_ref[...], wit[...], bi[...], wht[...],
                          bh[...]), 0.0)
    x2_o[...] = x2
    xl = jnp.dot(x2, awt[...], preferred_element_type=jnp.float32)
    xl_o[...] = xl
    as_o[...] = jnp.sum(xl * asrc[...], axis=1, keepdims=True)
    ad_o[...] = jnp.sum(xl * adst[...], axis=1, keepdims=True)


def _tce(y, s, x1, gbias, wit, bi, wht, bh, awt, asrc, adst):
    full = lambda s_: pl.BlockSpec(s_, lambda i: tuple(0 for _ in s_))
    return pl.pallas_call(
        _tce_body,
        grid=(NNB,),
        in_specs=[pl.BlockSpec((NC, NB, H), lambda i: (0, i, 0)),
                  pl.BlockSpec((NB, 1), lambda i: (i, 0)),
                  pl.BlockSpec((NB, H), lambda i: (i, 0)),
                  full((1, H)),
                  full((H, 3 * H)), full((1, 3 * H)),
                  full((H, 3 * H)), full((1, 3 * H)),
                  full((H, H)), full((1, H)), full((1, H))],
        out_specs=[pl.BlockSpec((NB, H), lambda i: (i, 0)),
                   pl.BlockSpec((NB, H), lambda i: (i, 0)),
                   pl.BlockSpec((NB, 1), lambda i: (i, 0)),
                   pl.BlockSpec((NB, 1), lambda i: (i, 0))],
        out_shape=[jax.ShapeDtypeStruct((N, H), jnp.float32),
                   jax.ShapeDtypeStruct((N, H), jnp.float32),
                   jax.ShapeDtypeStruct((N, 1), jnp.float32),
                   jax.ShapeDtypeStruct((N, 1), jnp.float32)],
    )(y, s, x1, gbias, wit, bi, wht, bh, awt, asrc, adst)


def _tcj_body(y2_ref, s2_ref, x2_ref, b_ref,
              abias, awit, abi, awht, abh, mwst, mats,
              xs_o, as_o, pooled_o):
    i = pl.program_id(0)
    y = y2_ref[0] + y2_ref[1]
    h2 = _elu(y / (s2_ref[...] + 1e-16) + abias[...])
    x3 = jnp.maximum(_gru(h2, x2_ref[...], awit[...], abi[...],
                          awht[...], abh[...]), 0.0)
    xs = jnp.dot(x3, mwst[...], preferred_element_type=jnp.float32)
    xs_o[...] = xs
    as_o[...] = jnp.sum(xs * mats[...], axis=1, keepdims=True)
    oh = (b_ref[...] == jax.lax.broadcasted_iota(jnp.int32, (1, G), 1)
          ).astype(jnp.float32)

    @pl.when(i == 0)
    def _():
        pooled_o[...] = jnp.zeros((G, H), jnp.float32)

    pooled_o[...] += lax.dot_general(oh, x3, (((0,), (0,)), ((), ())),
                                     preferred_element_type=jnp.float32,
                                     precision=lax.Precision.HIGHEST)


def _tcj(y2, s2, x2, batch2d, abias, awit, abi, awht, abh, mwst, mats):
    full = lambda s_: pl.BlockSpec(s_, lambda i: tuple(0 for _ in s_))
    return pl.pallas_call(
        _tcj_body,
        grid=(NNB,),
        in_specs=[pl.BlockSpec((NC, NB, H), lambda i: (0, i, 0)),
                  pl.BlockSpec((NB, 1), lambda i: (i, 0)),
                  pl.BlockSpec((NB, H), lambda i: (i, 0)),
                  pl.BlockSpec((NB, 1), lambda i: (i, 0)),
                  full((1, H)),
                  full((H, 3 * H)), full((1, 3 * H)),
                  full((H, 3 * H)), full((1, 3 * H)),
                  full((H, H)), full((1, H))],
        out_specs=[pl.BlockSpec((NB, H), lambda i: (i, 0)),
                   pl.BlockSpec((NB, 1), lambda i: (i, 0)),
                   full((G, H))],
        out_shape=[jax.ShapeDtypeStruct((N, H), jnp.float32),
                   jax.ShapeDtypeStruct((N, 1), jnp.float32),
                   jax.ShapeDtypeStruct((G, H), jnp.float32)],
    )(y2, s2, x2, batch2d, abias, awit, abi, awht, abh, mwst, mats)


def _tci_body(xs_ref, as_ref, b_ref, pooled_ref, t_ref, pf_ref,
              mwdt, matd, mbias, mwit, mbi, mwht, mbh,
              l2t, l2b, l3t, l3b, l4t, l4b,
              out_o, lg_s):
    iota_g = jax.lax.broadcasted_iota(jnp.int32, (1, G), 1)
    out = jnp.maximum(pooled_ref[...], 0.0)
    for _ in range(NT):
        od = jnp.dot(out, mwdt[...], preferred_element_type=jnp.float32)
        a_d = jnp.sum(od * matd[...], axis=1, keepdims=True)
        mseg = jnp.full((1, G), -1e30, jnp.float32)
        for b in range(N // 500):
            sl = pl.ds(b * 500, 500)
            oh = (b_ref[sl, :] == iota_g).astype(jnp.float32)
            adn = jnp.dot(oh, a_d, preferred_element_type=jnp.float32,
                          precision=lax.Precision.HIGHEST)
            lgb = _lrelu(as_ref[sl, :] + adn)
            lg_s[sl, :] = lgb
            masked = jnp.where(oh > 0, lgb, -1e30)
            mseg = jnp.maximum(mseg, jnp.max(masked, axis=0, keepdims=True))
        mseg = jnp.where(mseg > -1e29, mseg, 0.0)
        sg = jnp.zeros((G, 1), jnp.float32)
        wsum = jnp.zeros((G, H), jnp.float32)
        for b in range(N // 500):
            sl = pl.ds(b * 500, 500)
            oh = (b_ref[sl, :] == iota_g).astype(jnp.float32)
            mnode = jnp.dot(oh, mseg.T, preferred_element_type=jnp.float32,
                            precision=lax.Precision.HIGHEST)
            pb = jnp.exp(lg_s[sl, :] - mnode)
            sg = sg + lax.dot_general(oh, pb, (((0,), (0,)), ((), ())),
                                     preferred_element_type=jnp.float32,
                                     precision=lax.Precision.HIGHEST)
            wsum = wsum + lax.dot_general(
                oh, xs_ref[sl, :] * pb, (((0,), (0,)), ((), ())),
                preferred_element_type=jnp.float32,
                precision=lax.Precision.HIGHEST)
        h = _elu(wsum / (sg + 1e-16) + mbias[...])
        out = jnp.maximum(_gru(h, out, mwit[...], mbi[...], mwht[...],
                               mbh[...]), 0.0)
    z = jnp.concatenate([out, t_ref[...], pf_ref[...]], axis=1)
    z = jnp.maximum(jnp.dot(z, l2t[...],
                            preferred_element_type=jnp.float32) + l2b[...],
                    0.0)
    z = jnp.maximum(jnp.dot(z, l3t[...],
                            preferred_element_type=jnp.float32) + l3b[...],
                    0.0)
    out_o[...] = jnp.dot(z, l4t[...],
                         preferred_element_type=jnp.float32) + l4b[...]


def _tci(xs, as_, batch2d, pooled, t, pf, ws):
    return pl.pallas_call(
        _tci_body,
        out_shape=jax.ShapeDtypeStruct((G, 1), jnp.float32),
        scratch_shapes=[pltpu.VMEM((N, 1), jnp.float32)],
        compiler_params=pltpu.CompilerParams(
            vmem_limit_bytes=60 * 1024 * 1024),
    )(xs, as_, batch2d, pooled, t, pf, *ws)


def _stage_table(hbm, sh, sid):
    pltpu.sync_copy(hbm.at[pl.ds(sid * NPS, NPS)],
                    sh.at[pl.ds(sid * NPS, NPS)])

    @pl.when(sid == NS - 1)
    def _():
        pltpu.sync_copy(hbm.at[pl.ds(NS * NPS, N - NS * NPS)],
                        sh.at[pl.ds(NS * NPS, N - NS * NPS)])


def _stage_pack(hbm, sh, sid):
    @pl.when(sid < NP // 8)
    def _():
        pltpu.sync_copy(hbm.at[pl.ds(sid * 8, 8)], sh.at[pl.ds(sid * 8, 8)])


def _dump_table(acc_sh, y_hbm, cid, sid):
    pltpu.sync_copy(acc_sh.at[pl.ds(sid * NPS, NPS)],
                    y_hbm.at[cid, pl.ds(sid * NPS, NPS)])

    @pl.when(sid == NS - 1)
    def _():
        pltpu.sync_copy(acc_sh.at[pl.ds(NS * NPS, N - NS * NPS)],
                        y_hbm.at[cid, pl.ds(NS * NPS, N - NS * NPS)])

def _scb_body(u_hbm, rpad_hbm, src_hbm, dst_hbm, us_hbm, rd_hbm,
              utab_sh, rtab_v, sidx_v, didx_v, urows_v, rout_v, sem):
    cid = lax.axis_index("c")
    sid = lax.axis_index("s")
    wid = sid * NC + cid
    _stage_table(u_hbm, utab_sh, sid)
    pltpu.sync_copy(rpad_hbm, rtab_v)
    plsc.subcore_barrier()
    base = wid * EPW

    def chunk(c, _):
        cb = base + c * CH
        pltpu.sync_copy(src_hbm.at[pl.ds(cb, CH)], sidx_v)
        pltpu.sync_copy(dst_hbm.at[pl.ds(cb, CH)], didx_v)
        pltpu.async_copy(utab_sh.at[sidx_v], urows_v, sem).wait()
        pltpu.sync_copy(urows_v, us_hbm.at[pl.ds(cb, CH)])
        for g in range(CH // L):
            ii = didx_v[pl.ds(g * L, L)]
            rout_v[pl.ds(g * L, L)] = plsc.load_gather(rtab_v, [ii])
        pltpu.sync_copy(rout_v, rd_hbm.at[pl.ds(cb, CH)])
        return 0
    lax.fori_loop(0, NCHUNK, chunk, 0)


def _scb(u, rpad, src, dst):
    return pl.kernel(
        _scb_body,
        out_type=(jax.ShapeDtypeStruct((E, H), jnp.float32),
                  jax.ShapeDtypeStruct((E,), jnp.float32)),
        mesh=_mesh,
        compiler_params=pltpu.CompilerParams(needs_layout_passes=False),
        scratch_types=[
            pltpu.VMEM_SHARED((N, H), jnp.float32),
            pltpu.VMEM((NP * H,), jnp.float32),
            pltpu.VMEM((CH,), jnp.int32),
            pltpu.VMEM((CH,), jnp.int32),
            pltpu.VMEM((CH, H), jnp.float32),
            pltpu.VMEM((CH,), jnp.float32),
            pltpu.SemaphoreType.DMA,
        ],
    )(u, rpad, src, dst)


def _zero_spmem_slice(acc_sh, zbuf_v, sid):
    def zrow(i, _):
        def zcol(k, _):
            zbuf_v[i, pl.ds(k * L, L)] = jnp.zeros((L,), jnp.float32)
            return 0
        lax.fori_loop(0, H // L, zcol, 0)
        return 0
    lax.fori_loop(0, ZB, zrow, 0)

    def zcopy(k, _):
        pltpu.sync_copy(zbuf_v, acc_sh.at[pl.ds(sid * NPS + k * ZB, ZB)])
        return 0
    lax.fori_loop(0, NPS // ZB, zcopy, 0)

    @pl.when(sid == NS - 1)
    def _():
        pltpu.sync_copy(zbuf_v.at[pl.ds(0, N - NS * NPS)],
                        acc_sh.at[pl.ds(NS * NPS, N - NS * NPS)])


def _scale_rows(rows_v, p_v, ybuf_v):
    for g in range(CH // L):
        pvec = p_v[pl.ds(g * L, L)]
        for j in range(L):
            sp = pvec[jnp.full((L,), j, jnp.int32)]
            e = g * L + j
            for k in range(H // L):
                ybuf_v[e, pl.ds(k * L, L)] = rows_v[e, pl.ds(k * L, L)] * sp


def _scd_body(m_hbm, p_hbm, dst_hbm, y_hbm,
              acc_sh, didx_v, p_v, mrows_v, ybuf_v, zbuf_v, sem):
    cid = lax.axis_index("c")
    sid = lax.axis_index("s")
    wid = sid * NC + cid
    _zero_spmem_slice(acc_sh, zbuf_v, sid)
    plsc.subcore_barrier()
    base = wid * EPW

    def chunk(c, _):
        cb = base + c * CH
        pltpu.sync_copy(dst_hbm.at[pl.ds(cb, CH)], didx_v)
        pltpu.sync_copy(p_hbm.at[pl.ds(cb, CH)], p_v)
        pltpu.sync_copy(m_hbm.at[pl.ds(cb, CH)], mrows_v)
        _scale_rows(mrows_v, p_v, ybuf_v)
        pltpu.sync_copy(ybuf_v, acc_sh.at[didx_v], add=True)
        return 0
    lax.fori_loop(0, NCHUNK, chunk, 0)
    plsc.subcore_barrier()
    _dump_table(acc_sh, y_hbm, cid, sid)


def _scd(m, pvals, dst):
    return pl.kernel(
        _scd_body,
        out_type=jax.ShapeDtypeStruct((NC, N, H), jnp.float32),
        mesh=_mesh,
        compiler_params=pltpu.CompilerParams(needs_layout_passes=False),
        scratch_types=[
            pltpu.VMEM_SHARED((N, H), jnp.float32),
            pltpu.VMEM((CH,), jnp.int32),
            pltpu.VMEM((CH,), jnp.float32),
            pltpu.VMEM((CH, H), jnp.float32),
            pltpu.VMEM((CH, H), jnp.float32),
            pltpu.VMEM((ZB, H), jnp.float32),
            pltpu.SemaphoreType.DMA,
        ],
    )(m, pvals, dst)


def _scf_body(apad_hbm, dpad_hbm, src_hbm, dst_hbm, lg_hbm,
              atab_v, dtab_v, sidx_v, didx_v, lout_v):
    cid = lax.axis_index("c")
    sid = lax.axis_index("s")
    wid = sid * NC + cid
    pltpu.sync_copy(apad_hbm, atab_v)
    pltpu.sync_copy(dpad_hbm, dtab_v)
    base = wid * EPW

    def chunk(c, _):
        cb = base + c * CH
        pltpu.sync_copy(src_hbm.at[pl.ds(cb, CH)], sidx_v)
        pltpu.sync_copy(dst_hbm.at[pl.ds(cb, CH)], didx_v)
        for g in range(CH // L):
            va = plsc.load_gather(atab_v, [sidx_v[pl.ds(g * L, L)]])
            vd = plsc.load_gather(dtab_v, [didx_v[pl.ds(g * L, L)]])
            lout_v[pl.ds(g * L, L)] = _lrelu(va + vd)
        pltpu.sync_copy(lout_v, lg_hbm.at[pl.ds(cb, CH)])
        return 0
    lax.fori_loop(0, NCHUNK, chunk, 0)


def _scf(apad, dpad, src, dst):
    return pl.kernel(
        _scf_body,
        out_type=jax.ShapeDtypeStruct((E,), jnp.float32),
        mesh=_mesh,
        compiler_params=pltpu.CompilerParams(needs_layout_passes=False),
        scratch_types=[
            pltpu.VMEM((NP * H,), jnp.float32),
            pltpu.VMEM((NP * H,), jnp.float32),
            pltpu.VMEM((CH,), jnp.int32),
            pltpu.VMEM((CH,), jnp.int32),
            pltpu.VMEM((CH,), jnp.float32),
        ],
    )(apad, dpad, src, dst)


def _sch_body(xl_hbm, p_hbm, src_hbm, dst_hbm, y_hbm,
              acc_sh, sidx_v, didx_v, p_v, xrows_v, ybuf_v, zbuf_v, sem):
    cid = lax.axis_index("c")
    sid = lax.axis_index("s")
    wid = sid * NC + cid
    _zero_spmem_slice(acc_sh, zbuf_v, sid)
    plsc.subcore_barrier()
    base = wid * EPW

    def chunk(c, _):
        cb = base + c * CH
        pltpu.sync_copy(src_hbm.at[pl.ds(cb, CH)], sidx_v)
        pltpu.sync_copy(dst_hbm.at[pl.ds(cb, CH)], didx_v)
        pltpu.sync_copy(p_hbm.at[pl.ds(cb, CH)], p_v)
        pltpu.async_copy(xl_hbm.at[sidx_v], xrows_v, sem).wait()
        _scale_rows(xrows_v, p_v, ybuf_v)
        pltpu.sync_copy(ybuf_v, acc_sh.at[didx_v], add=True)
        return 0
    lax.fori_loop(0, NCHUNK, chunk, 0)
    plsc.subcore_barrier()
    _dump_table(acc_sh, y_hbm, cid, sid)


def _sch(xl, pvals, src, dst):
    return pl.kernel(
        _sch_body,
        out_type=jax.ShapeDtypeStruct((NC, N, H), jnp.float32),
        mesh=_mesh,
        compiler_params=pltpu.CompilerParams(needs_layout_passes=False),
        scratch_types=[
            pltpu.VMEM_SHARED((N, H), jnp.float32),
            pltpu.VMEM((CH,), jnp.int32),
            pltpu.VMEM((CH,), jnp.int32),
            pltpu.VMEM((CH,), jnp.float32),
            pltpu.VMEM((CH, H), jnp.float32),
            pltpu.VMEM((CH, H), jnp.float32),
            pltpu.VMEM((ZB, H), jnp.float32),
            pltpu.SemaphoreType.DMA,
        ],
    )(xl, pvals, src, dst)


def _pack_scalar(v):
    return jnp.pad(v.reshape(-1), (0, NP * H - N))


def kernel(x, edge_attr, t, p, params, edge_index, batch):
    pr = params
    src = edge_index[0]
    dst = edge_index[1]

    x1, r = _tca(x, pr['lin1_w'].T, pr['lin1_b'][None],
                 pr['g_att_r'][None])
    xjsrc, rdst = _scb(x1, _pack_scalar(r), src, dst)
    m, logit, mxvec = _tcc(xjsrc, edge_attr, rdst[:, None],
                           pr['g_lin1_w'].T, pr['g_lin2_w'].T,
                           pr['g_att_l'][None])
    pvals, spack = _tcseg(logit, dst[:, None], mxvec)
    y = _scd(m, pvals.reshape(-1), dst)
    s = spack.reshape(-1)[:N][:, None]
    x2, xl, as_, ad_ = _tce(
        y, s, x1, pr['g_bias'][None],
        pr['gru1_wi'].T, pr['gru1_bi'][None], pr['gru1_wh'].T,
        pr['gru1_bh'][None], pr['a_w'].T, pr['a_att_src'][None],
        pr['a_att_dst'][None])
    lg2 = _scf(_pack_scalar(as_), _pack_scalar(ad_), src, dst)
    mx2vec = _tcmax(lg2[:, None])
    p2, s2pack = _tcseg(lg2[:, None], dst[:, None], mx2vec)
    y2 = _sch(xl, p2.reshape(-1), src, dst)
    s2 = s2pack.reshape(-1)[:N][:, None]
    xs, a_s, pooled = _tcj(
        y2, s2, x2, batch[:, None], pr['a_bias'][None],
        pr['agru_wi'].T, pr['agru_bi'][None], pr['agru_wh'].T,
        pr['agru_bh'][None], pr['m_w_src'].T, pr['m_att_src'][None])
    ws = [pr['m_w_dst'].T, pr['m_att_dst'][None], pr['m_bias'][None],
          pr['mgru_wi'].T, pr['mgru_bi'][None], pr['mgru_wh'].T,
          pr['mgru_bh'][None],
          pr['lin2_w'].T, pr['lin2_b'][None],
          pr['lin3_w'].T, pr['lin3_b'][None],
          pr['lin4_w'].T, pr['lin4_b'][None]]
    out = _tci(xs, a_s, batch[:, None], pooled, t, p, ws)
    return out.reshape(-1)
